# baseline (speedup 1.0000x reference)
import jax
import jax.numpy as jnp
from jax import lax
from jax.experimental import pallas as pl
from jax.experimental.pallas import tpu as pltpu

NUM_INPUTS = 8192
D_MODEL = 4096
_R = 64
_C = 128


def _body(w_ref, g_ref, x_ref, out_ref, sem):
    v = w_ref[...] + g_ref[...]
    m = jnp.max(v)
    flat = (lax.broadcasted_iota(jnp.int32, (_R, _C), 0) * _C
            + lax.broadcasted_iota(jnp.int32, (_R, _C), 1))
    idx = jnp.min(jnp.where(v == m, flat, 2**31 - 1))
    copy = pltpu.make_async_copy(x_ref.at[pl.ds(idx, 1)], out_ref, sem)
    copy.start()
    copy.wait()


def _body_hbm(w_ref, g_ref, x_ref, out_ref, sem):
    v = w_ref[...] + g_ref[...]
    m = jnp.max(v)
    flat = (lax.broadcasted_iota(jnp.int32, (_R, _C), 0) * _C
            + lax.broadcasted_iota(jnp.int32, (_R, _C), 1))
    idx = jnp.min(jnp.where(v == m, flat, 2**31 - 1))
    copy = pltpu.make_async_copy(x_ref.at[pl.ds(idx, 1)], out_ref, sem)
    copy.start()
    copy.wait()


def kernel(x, weights):
    gkey = jax.random.key(42)
    u = jax.random.uniform(gkey, weights.shape, dtype=weights.dtype,
                           minval=1e-10, maxval=1.0)
    gumbels = -jnp.log(-jnp.log(u))
    w2 = weights.reshape(_R, _C)
    g2 = gumbels.reshape(_R, _C)
    return pl.pallas_call(
        _body_hbm,
        grid=(1,),
        in_specs=[
            pl.BlockSpec((_R, _C), lambda i: (0, 0)),
            pl.BlockSpec((_R, _C), lambda i: (0, 0)),
            pl.BlockSpec(memory_space=pltpu.MemorySpace.HBM),
        ],
        out_specs=pl.BlockSpec(memory_space=pltpu.MemorySpace.HBM),
        out_shape=jax.ShapeDtypeStruct((1, D_MODEL), jnp.float32),
        scratch_shapes=[pltpu.SemaphoreType.DMA],
    )(w2, g2, x)


# speedup vs baseline: 1.0735x; 1.0735x over previous
import jax
import jax.numpy as jnp
from jax import lax
from jax.experimental import pallas as pl
from jax.experimental.pallas import tpu as pltpu

NUM_INPUTS = 8192
D_MODEL = 4096
_R = 64
_C = 128


def _body(w_hbm, g_hbm, x_hbm, out_ref, w_v, g_v, sem_w, sem_g, sem_x):
    cp_w = pltpu.make_async_copy(w_hbm, w_v, sem_w)
    cp_g = pltpu.make_async_copy(g_hbm, g_v, sem_g)
    cp_w.start()
    cp_g.start()
    cp_w.wait()
    cp_g.wait()
    v = w_v[...] + g_v[...]
    m = jnp.max(v)
    flat = (lax.broadcasted_iota(jnp.int32, (_R, _C), 0) * _C
            + lax.broadcasted_iota(jnp.int32, (_R, _C), 1))
    idx = jnp.min(jnp.where(v == m, flat, 2**31 - 1))
    copy = pltpu.make_async_copy(x_hbm.at[pl.ds(idx, 1)], out_ref, sem_x)
    copy.start()
    copy.wait()


def kernel(x, weights):
    gkey = jax.random.key(42)
    u = jax.random.uniform(gkey, weights.shape, dtype=weights.dtype,
                           minval=1e-10, maxval=1.0)
    gumbels = -jnp.log(-jnp.log(u))
    w2 = weights.reshape(_R, _C)
    g2 = gumbels.reshape(_R, _C)
    return pl.pallas_call(
        _body,
        grid=(1,),
        in_specs=[
            pl.BlockSpec(memory_space=pltpu.MemorySpace.HBM),
            pl.BlockSpec(memory_space=pltpu.MemorySpace.HBM),
            pl.BlockSpec(memory_space=pltpu.MemorySpace.HBM),
        ],
        out_specs=pl.BlockSpec((1, D_MODEL), lambda i: (0, 0)),
        out_shape=jax.ShapeDtypeStruct((1, D_MODEL), jnp.float32),
        scratch_shapes=[
            pltpu.VMEM((_R, _C), jnp.float32),
            pltpu.VMEM((_R, _C), jnp.float32),
            pltpu.SemaphoreType.DMA,
            pltpu.SemaphoreType.DMA,
            pltpu.SemaphoreType.DMA,
        ],
    )(w2, g2, x)
